# baseline (speedup 1.0000x reference)
import functools

import jax
import jax.numpy as jnp
from jax import lax
from jax.experimental import pallas as pl
from jax.experimental.pallas import tpu as pltpu
from jax.experimental.pallas import tpu_sc as plsc

f32 = jnp.float32
i32 = jnp.int32

N = 10000
D = 64
DP = 128
K = 20
E = N * K
GD = 4
NW = 32
CH = 128
EPAD = 204800
NCH = EPAD // (NW * CH)
NPAD = 10240
NROW = NPAD // 16
CPAD = 10240
RB = 256
EB0 = 2048
EB1 = 4000
BIGF = 3.0e38
BIGI = 2 ** 30


def _mesh():
    return plsc.VectorSubcoreMesh(core_axis_name="c", subcore_axis_name="s")


def _rpad(w):
    return jnp.concatenate([w, jnp.zeros((DP - D,) + w.shape[1:], f32)])


def _cpad(w):
    return jnp.concatenate([w, jnp.zeros(w.shape[:-1] + (DP - D,), f32)], axis=-1)


def _sc_gather(table, idx3d):
    @functools.partial(
        pl.kernel,
        out_type=jax.ShapeDtypeStruct((EPAD, DP), f32),
        mesh=_mesh(),
        scratch_types=[
            pltpu.VMEM((NCH, CH), i32),
            pltpu.VMEM((CH, DP), f32),
            pltpu.SemaphoreType.DMA,
        ],
    )
    def k(table_h, idx_h, out_h, idx_v, rows_v, sem):
        cid = lax.axis_index("c")
        sid = lax.axis_index("s")
        wid = sid * 2 + cid
        rowbase = wid * NCH
        pltpu.sync_copy(idx_h.at[wid], idx_v)

        def body(c, carry):
            pltpu.async_copy(table_h.at[idx_v.at[c]], rows_v, sem).wait()
            pltpu.sync_copy(rows_v, out_h.at[pl.ds((rowbase + c) * CH, CH)])
            return carry

        lax.fori_loop(0, NCH, body, 0)

    return k(table, idx3d)


def _sc_scatter_add(edges, idx3d):
    nz = NROW // CH

    @functools.partial(
        pl.kernel,
        out_type=jax.ShapeDtypeStruct((2, NPAD, DP), f32),
        mesh=_mesh(),
        scratch_types=[
            pltpu.VMEM((NCH, CH), i32),
            [pltpu.VMEM((CH, DP), f32)] * 2,
            pltpu.VMEM_SHARED((NPAD, DP), f32),
            [pltpu.SemaphoreType.DMA] * 2,
            [pltpu.SemaphoreType.DMA] * 2,
        ],
    )
    def k(edges_h, idx_h, out_h, idx_v, ebufs, shared, lsems, ssems):
        cid = lax.axis_index("c")
        sid = lax.axis_index("s")
        wid = sid * 2 + cid
        rows_v = ebufs[0]

        def zbody(i, carry):
            r = i // (DP // 16)
            c = (i % (DP // 16)) * 16
            rows_v[r, pl.ds(c, 16)] = jnp.zeros((16,), f32)
            return carry

        lax.fori_loop(0, CH * (DP // 16), zbody, 0)
        for z in range(nz):
            pltpu.sync_copy(rows_v, shared.at[pl.ds(sid * NROW + z * CH, CH)])
        plsc.subcore_barrier()
        rowbase = wid * NCH
        pltpu.sync_copy(idx_h.at[wid], idx_v)

        def fire_load(c, b):
            pltpu.async_copy(edges_h.at[pl.ds((rowbase + c) * CH, CH)],
                             ebufs[b], lsems[b])

        fire_load(0, 0)

        def step(c, b):
            pltpu.make_async_copy(edges_h.at[pl.ds(rowbase * CH, CH)],
                                  ebufs[b], lsems[b]).wait()
            pltpu.async_copy(ebufs[b], shared.at[idx_v.at[c]], ssems[b],
                             add=True)

            @pl.when(c >= 1)
            def _():
                pltpu.make_async_copy(ebufs[1 - b], shared.at[idx_v.at[0]],
                                      ssems[1 - b]).wait()

            @pl.when(c + 1 < NCH)
            def _():
                fire_load(c + 1, 1 - b)

        def body(cc, carry):
            for b in range(2):
                step(cc * 2 + b, b)
            return carry

        lax.fori_loop(0, NCH // 2, body, 0)
        pltpu.make_async_copy(ebufs[1], shared.at[idx_v.at[0]],
                              ssems[1]).wait()
        plsc.subcore_barrier()
        for z in range(nz):
            pltpu.sync_copy(shared.at[pl.ds(sid * NROW + z * CH, CH)], rows_v)
            pltpu.sync_copy(rows_v,
                            out_h.at[cid, pl.ds(sid * NROW + z * CH, CH)])

    return k(edges, idx3d)


def _full(shape):
    return pl.BlockSpec(shape, lambda *_: tuple(0 for _ in shape))


def _tc_embed(x, mlp):
    (w0, b0), (w1, b1), (w2, b2) = mlp
    w2p, b2p = _cpad(w2), _cpad(b2.reshape(1, -1))

    def kern(x_r, w0_r, b0_r, w1_r, b1_r, w2_r, b2_r, o_r):
        h = jnp.dot(x_r[...], w0_r[...], preferred_element_type=f32) + b0_r[...]
        h = jnp.maximum(h, 0.0)
        h = jnp.dot(h, w1_r[...], preferred_element_type=f32) + b1_r[...]
        h = jnp.maximum(h, 0.0)
        o_r[...] = jnp.dot(h, w2_r[...], preferred_element_type=f32) + b2_r[...]

    return pl.pallas_call(
        kern, out_shape=jax.ShapeDtypeStruct((N, DP), f32),
    )(x, w0, b0.reshape(1, -1), w1, b1.reshape(1, -1), w2p, b2p)


def _tc_edge0(sent, recv, g, mlp, ln):
    (w0, b0), (w1, b1), (w2, b2) = mlp
    wd, wg = _rpad(w0[:D]), w0[D:]
    w2p, b2p = _cpad(w2), _cpad(b2.reshape(1, -1))
    scale, bias = ln

    def kern(s_r, r_r, g_r, wd_r, wg_r, b0_r, w1_r, b1_r, w2_r, b2_r,
             sc_r, bi_r, raw_o, ln_o):
        dif = s_r[...] - r_r[...]
        gv = jnp.dot(g_r[...], wg_r[...], preferred_element_type=f32) + b0_r[...]
        h = jnp.maximum(jnp.dot(dif, wd_r[...], preferred_element_type=f32) + gv, 0.0)
        h = jnp.maximum(jnp.dot(h, w1_r[...], preferred_element_type=f32) + b1_r[...], 0.0)
        e = jnp.dot(h, w2_r[...], preferred_element_type=f32) + b2_r[...]
        raw_o[...] = e
        e64 = e[:, :D]
        mu = jnp.mean(e64, axis=-1, keepdims=True)
        var = jnp.mean((e64 - mu) ** 2, axis=-1, keepdims=True)
        ln_o[...] = (e64 - mu) * lax.rsqrt(var + 1e-6) * sc_r[...] + bi_r[...]

    blkp = pl.BlockSpec((EB0, DP), lambda i: (i, 0))
    blk = pl.BlockSpec((EB0, D), lambda i: (i, 0))
    return pl.pallas_call(
        kern,
        grid=(EPAD // EB0,),
        in_specs=[blkp, blkp, _full((1, GD)), _full((DP, D)), _full((GD, D)),
                  _full((1, D)), _full((D, D)), _full((1, D)), _full((D, DP)),
                  _full((1, DP)), _full((1, D)), _full((1, D))],
        out_specs=[blkp, blk],
        out_shape=[jax.ShapeDtypeStruct((EPAD, DP), f32),
                   jax.ShapeDtypeStruct((EPAD, D), f32)],
    )(sent, recv, g, wd, wg, b0.reshape(1, -1), w1, b1.reshape(1, -1),
      w2p, b2p, scale.reshape(1, -1), bias.reshape(1, -1))


def _tc_edge1(eprev, recv, nodes, g, mlp):
    (w0, b0), (w1, b1), (w2, b2) = mlp
    we, wd, wg = w0[:D], _rpad(w0[D:2 * D]), w0[2 * D:]
    w2p, b2p = _cpad(w2), _cpad(b2.reshape(1, -1))
    nb = EB1 // K

    def kern(e_r, r_r, n_r, g_r, we_r, wd_r, wg_r, b0_r, w1_r, b1_r,
             w2_r, b2_r, raw_o):
        sent = jnp.broadcast_to(n_r[...][:, None, :], (nb, K, DP)).reshape(EB1, DP)
        dif = sent - r_r[...]
        gv = jnp.dot(g_r[...], wg_r[...], preferred_element_type=f32) + b0_r[...]
        h = (jnp.dot(e_r[...], we_r[...], preferred_element_type=f32)
             + jnp.dot(dif, wd_r[...], preferred_element_type=f32) + gv)
        h = jnp.maximum(h, 0.0)
        h = jnp.maximum(jnp.dot(h, w1_r[...], preferred_element_type=f32) + b1_r[...], 0.0)
        raw_o[...] = jnp.dot(h, w2_r[...], preferred_element_type=f32) + b2_r[...]

    blkp = pl.BlockSpec((EB1, DP), lambda i: (i, 0))
    blk = pl.BlockSpec((EB1, D), lambda i: (i, 0))
    return pl.pallas_call(
        kern,
        grid=(E // EB1,),
        in_specs=[blk, blkp, pl.BlockSpec((nb, DP), lambda i: (i, 0)),
                  _full((1, GD)), _full((D, D)), _full((DP, D)), _full((GD, D)),
                  _full((1, D)), _full((D, D)), _full((1, D)), _full((D, DP)),
                  _full((1, DP))],
        out_specs=blkp,
        out_shape=jax.ShapeDtypeStruct((EPAD, DP), f32),
    )(eprev, recv, nodes, g, we, wd, wg, b0.reshape(1, -1), w1,
      b1.reshape(1, -1), w2p, b2p)


def _node_mlp_body(nodes, recvd, g_r, wn_r, wr_r, wg_r, b0_r, w1_r, b1_r,
                   w2_r, b2_r, sc_r, bi_r):
    gv = jnp.dot(g_r[...], wg_r[...], preferred_element_type=f32) + b0_r[...]
    h = (jnp.dot(nodes, wn_r[...], preferred_element_type=f32)
         + jnp.dot(recvd, wr_r[...], preferred_element_type=f32) + gv)
    h = jnp.maximum(h, 0.0)
    h = jnp.maximum(jnp.dot(h, w1_r[...], preferred_element_type=f32) + b1_r[...], 0.0)
    new = jnp.dot(h, w2_r[...], preferred_element_type=f32) + b2_r[...]
    x = nodes[:, :D] + new
    mu = jnp.mean(x, axis=-1, keepdims=True)
    var = jnp.mean((x - mu) ** 2, axis=-1, keepdims=True)
    return (x - mu) * lax.rsqrt(var + 1e-6) * sc_r[...] + bi_r[...]


def _tc_node(nodes, p0, p1, g, mlp, ln):
    (w0, b0), (w1, b1), (w2, b2) = mlp
    wn, wr, wg = _rpad(w0[:D]), _rpad(w0[D:2 * D]), w0[2 * D:]
    scale, bias = ln

    def kern(n_r, p0_r, p1_r, g_r, wn_r, wr_r, wg_r, b0_r, w1_r, b1_r,
             w2_r, b2_r, sc_r, bi_r, o_r):
        recvd = p0_r[...] + p1_r[...]
        y = _node_mlp_body(n_r[...], recvd, g_r, wn_r, wr_r, wg_r, b0_r,
                           w1_r, b1_r, w2_r, b2_r, sc_r, bi_r)
        o_r[...] = jnp.concatenate([y, jnp.zeros_like(y)], axis=-1)

    return pl.pallas_call(
        kern,
        grid=(1,),
        in_specs=[_full((N, DP)), _full((N, DP)), _full((N, DP)),
                  _full((1, GD)), _full((DP, D)), _full((DP, D)),
                  _full((GD, D)), _full((1, D)), _full((D, D)), _full((1, D)),
                  _full((D, D)), _full((1, D)), _full((1, D)), _full((1, D))],
        out_specs=pl.BlockSpec((N, DP), lambda i: (0, 0)),
        out_shape=jax.ShapeDtypeStruct((CPAD, DP), f32),
    )(nodes, p0, p1, g, wn, wr, wg, b0.reshape(1, -1), w1, b1.reshape(1, -1),
      w2, b2.reshape(1, -1), scale.reshape(1, -1), bias.reshape(1, -1))


def _tc_node_decode(nodes, p0, p1, g, mlp, ln, dec):
    (w0, b0), (w1, b1), (w2, b2) = mlp
    wn, wr, wg = _rpad(w0[:D]), _rpad(w0[D:2 * D]), w0[2 * D:]
    scale, bias = ln
    (dw0, db0), (dw1, db1), (dw2, db2) = dec

    def kern(n_r, p0_r, p1_r, g_r, wn_r, wr_r, wg_r, b0_r, w1_r, b1_r,
             w2_r, b2_r, sc_r, bi_r, dw0_r, db0_r, dw1_r, db1_r, dw2_r,
             db2_r, o_r):
        recvd = p0_r[...] + p1_r[...]
        x = _node_mlp_body(n_r[...], recvd, g_r, wn_r, wr_r, wg_r, b0_r,
                           w1_r, b1_r, w2_r, b2_r, sc_r, bi_r)
        h = jnp.maximum(jnp.dot(x, dw0_r[...], preferred_element_type=f32) + db0_r[...], 0.0)
        h = jnp.maximum(jnp.dot(h, dw1_r[...], preferred_element_type=f32) + db1_r[...], 0.0)
        o_r[...] = jnp.dot(h, dw2_r[...], preferred_element_type=f32) + db2_r[...]

    return pl.pallas_call(
        kern, out_shape=jax.ShapeDtypeStruct((N, 3), f32),
    )(nodes, p0, p1, g, wn, wr, wg, b0.reshape(1, -1), w1, b1.reshape(1, -1),
      w2, b2.reshape(1, -1), scale.reshape(1, -1), bias.reshape(1, -1),
      dw0, db0.reshape(1, -1), dw1, db1.reshape(1, -1), dw2, db2.reshape(1, -1))


NCHUNK = CPAD // CH
TOPT = 6


def _tc_knn(xpad):
    def kern(xa_r, xb_r, o_r):
        xa = xa_r[...]
        xb = xb_r[...]
        sqc = jnp.sum(xa * xa, axis=-1, keepdims=True)
        sqr = lax.dot_general(jnp.ones((1, DP), f32), xb * xb,
                              (((1,), (1,)), ((), ())),
                              preferred_element_type=f32)
        m = lax.dot_general(xa, xb, (((1,), (1,)), ((), ())),
                            preferred_element_type=f32)
        ri = lax.broadcasted_iota(i32, (CPAD, RB), 0)
        d2 = jnp.where(ri < N, sqc + sqr - 2.0 * m, BIGF)

        d3 = d2.reshape(NCHUNK, CH, RB)
        li = lax.broadcasted_iota(i32, (NCHUNK, CH, RB), 1)
        gb = lax.broadcasted_iota(i32, (NCHUNK, RB), 0) * CH
        ys, gs = [], []
        for _ in range(TOPT):
            mn = jnp.min(d3, axis=1)
            am = jnp.min(jnp.where(d3 <= mn[:, None, :], li, BIGI), axis=1)
            ys.append(mn)
            gs.append(gb + am)
            d3 = jnp.where(li == am[:, None, :], BIGF, d3)
        y = jnp.concatenate(ys, axis=0)
        gi = jnp.concatenate(gs, axis=0)

        last = 0
        for t in range(K):
            mn = jnp.min(y, axis=0, keepdims=True)
            ix = jnp.min(jnp.where(y <= mn, gi, BIGI), axis=0, keepdims=True)
            o_r[pl.ds(t, 1), :] = ix
            sel = gi == ix
            y = jnp.where(sel, BIGF, y)
            gi = jnp.where(sel, BIGI, gi)
        exhausted = jnp.any(gi[NCHUNK * (TOPT - 1):, :] == BIGI)

        @pl.when(exhausted)
        def _():
            dd = d2
            for t in range(K):
                mn = jnp.min(dd, axis=0, keepdims=True)
                ix = jnp.min(jnp.where(dd <= mn, ri, BIGI), axis=0,
                             keepdims=True)
                o_r[pl.ds(t, 1), :] = ix
                dd = jnp.where(ri == ix, BIGF, dd)

    return pl.pallas_call(
        kern,
        grid=(CPAD // RB,),
        in_specs=[_full((CPAD, DP)), pl.BlockSpec((RB, DP), lambda i: (i, 0))],
        out_specs=pl.BlockSpec((K, RB), lambda i: (0, i)),
        out_shape=jax.ShapeDtypeStruct((K, CPAD), i32),
    )(xpad, xpad)


def _pad_idx(idx, fill):
    idx = jnp.concatenate(
        [idx.astype(i32), jnp.full((EPAD - idx.shape[0],), fill, i32)])
    return idx.reshape(NW, NCH, CH)


def kernel(nodes, globals_, senders, receivers, params):
    g = globals_.reshape(1, GD)

    x = _tc_embed(nodes, params["embed_node"])

    sidx = _pad_idx(senders, 0)
    ridx = _pad_idx(receivers, 0)
    scat_idx0 = _pad_idx(receivers, N)
    sent = _sc_gather(x, sidx)
    recv = _sc_gather(x, ridx)
    e_raw, e_ln = _tc_edge0(sent, recv, g, params["edge_0"], params["ln_0"])
    parts = _sc_scatter_add(e_raw, scat_idx0)
    xpad = _tc_node(x, parts[0, :N], parts[1, :N], g, params["node_0"],
                    params["ln_0"])
    x = xpad[:N]

    knn = _tc_knn(xpad)[:, :N].T

    recv_flat = knn.reshape(-1)
    ridx1 = _pad_idx(recv_flat, 0)
    scat_idx1 = _pad_idx(recv_flat, N)
    recv1 = _sc_gather(xpad, ridx1)
    e1_raw = _tc_edge1(e_ln, recv1, x, g, params["edge_1"])
    parts1 = _sc_scatter_add(e1_raw, scat_idx1)
    return _tc_node_decode(x, parts1[0, :N], parts1[1, :N], g,
                           params["node_1"], params["ln_1"],
                           params["decode_node"])


# speedup vs baseline: 1.4083x; 1.4083x over previous
import functools

import jax
import jax.numpy as jnp
from jax import lax
from jax.experimental import pallas as pl
from jax.experimental.pallas import tpu as pltpu
from jax.experimental.pallas import tpu_sc as plsc

f32 = jnp.float32
i32 = jnp.int32

N = 10000
D = 64
DP = 128
K = 20
E = N * K
GD = 4
NW = 32
CH = 128
EPAD = 200704
NCH = EPAD // (NW * CH)
NPAD = 10240
NROW = NPAD // 16
CPAD = 10240
RB = 256
EB0 = 2048
EB1 = 4000
BIGF = 3.0e38
BIGI = 2 ** 30


def _mesh():
    return plsc.VectorSubcoreMesh(core_axis_name="c", subcore_axis_name="s")


def _rpad(w):
    return jnp.concatenate([w, jnp.zeros((DP - D,) + w.shape[1:], f32)])


def _cpad(w):
    return jnp.concatenate([w, jnp.zeros(w.shape[:-1] + (DP - D,), f32)], axis=-1)


def _sc_gather(table, idx3d):
    @functools.partial(
        pl.kernel,
        out_type=jax.ShapeDtypeStruct((EPAD, DP), f32),
        mesh=_mesh(),
        scratch_types=[
            pltpu.VMEM((NCH, CH), i32),
            pltpu.VMEM((CH, DP), f32),
            pltpu.SemaphoreType.DMA,
        ],
    )
    def k(table_h, idx_h, out_h, idx_v, rows_v, sem):
        cid = lax.axis_index("c")
        sid = lax.axis_index("s")
        wid = sid * 2 + cid
        rowbase = wid * NCH
        pltpu.sync_copy(idx_h.at[wid], idx_v)

        def body(c, carry):
            pltpu.async_copy(table_h.at[idx_v.at[c]], rows_v, sem).wait()
            pltpu.sync_copy(rows_v, out_h.at[pl.ds((rowbase + c) * CH, CH)])
            return carry

        lax.fori_loop(0, NCH, body, 0)

    return k(table, idx3d)


def _sc_scatter_add(edges, idx3d):
    nz = NROW // CH

    @functools.partial(
        pl.kernel,
        out_type=jax.ShapeDtypeStruct((2, NPAD, DP), f32),
        mesh=_mesh(),
        scratch_types=[
            pltpu.VMEM((NCH, CH), i32),
            pltpu.VMEM((CH, DP), f32),
            pltpu.VMEM_SHARED((NPAD, DP), f32),
            pltpu.SemaphoreType.DMA,
        ],
    )
    def k(edges_h, idx_h, out_h, idx_v, rows_v, shared, sem):
        cid = lax.axis_index("c")
        sid = lax.axis_index("s")
        wid = sid * 2 + cid

        def zbody(i, carry):
            r = i // (DP // 16)
            c = (i % (DP // 16)) * 16
            rows_v[r, pl.ds(c, 16)] = jnp.zeros((16,), f32)
            return carry

        lax.fori_loop(0, CH * (DP // 16), zbody, 0)
        for z in range(nz):
            pltpu.sync_copy(rows_v, shared.at[pl.ds(sid * NROW + z * CH, CH)])
        plsc.subcore_barrier()
        rowbase = wid * NCH
        pltpu.sync_copy(idx_h.at[wid], idx_v)

        def body(c, carry):
            pltpu.sync_copy(edges_h.at[pl.ds((rowbase + c) * CH, CH)], rows_v)
            pltpu.sync_copy(rows_v, shared.at[idx_v.at[c]], add=True)
            return carry

        lax.fori_loop(0, NCH, body, 0)
        plsc.subcore_barrier()
        for z in range(nz):
            pltpu.sync_copy(shared.at[pl.ds(sid * NROW + z * CH, CH)], rows_v)
            pltpu.sync_copy(rows_v,
                            out_h.at[cid, pl.ds(sid * NROW + z * CH, CH)])

    return k(edges, idx3d)


def _full(shape):
    return pl.BlockSpec(shape, lambda *_: tuple(0 for _ in shape))


def _tc_embed(x, mlp):
    (w0, b0), (w1, b1), (w2, b2) = mlp
    w2p, b2p = _cpad(w2), _cpad(b2.reshape(1, -1))

    def kern(x_r, w0_r, b0_r, w1_r, b1_r, w2_r, b2_r, o_r):
        h = jnp.dot(x_r[...], w0_r[...], preferred_element_type=f32) + b0_r[...]
        h = jnp.maximum(h, 0.0)
        h = jnp.dot(h, w1_r[...], preferred_element_type=f32) + b1_r[...]
        h = jnp.maximum(h, 0.0)
        o_r[...] = jnp.dot(h, w2_r[...], preferred_element_type=f32) + b2_r[...]

    return pl.pallas_call(
        kern, out_shape=jax.ShapeDtypeStruct((N, DP), f32),
    )(x, w0, b0.reshape(1, -1), w1, b1.reshape(1, -1), w2p, b2p)


def _tc_edge0(sent, recv, g, mlp, ln):
    (w0, b0), (w1, b1), (w2, b2) = mlp
    wd, wg = _rpad(w0[:D]), w0[D:]
    w2p, b2p = _cpad(w2), _cpad(b2.reshape(1, -1))
    scale, bias = ln

    def kern(s_r, r_r, g_r, wd_r, wg_r, b0_r, w1_r, b1_r, w2_r, b2_r,
             sc_r, bi_r, raw_o, ln_o):
        dif = s_r[...] - r_r[...]
        gv = jnp.dot(g_r[...], wg_r[...], preferred_element_type=f32) + b0_r[...]
        h = jnp.maximum(jnp.dot(dif, wd_r[...], preferred_element_type=f32) + gv, 0.0)
        h = jnp.maximum(jnp.dot(h, w1_r[...], preferred_element_type=f32) + b1_r[...], 0.0)
        e = jnp.dot(h, w2_r[...], preferred_element_type=f32) + b2_r[...]
        raw_o[...] = e
        e64 = e[:, :D]
        mu = jnp.mean(e64, axis=-1, keepdims=True)
        var = jnp.mean((e64 - mu) ** 2, axis=-1, keepdims=True)
        ln_o[...] = (e64 - mu) * lax.rsqrt(var + 1e-6) * sc_r[...] + bi_r[...]

    blkp = pl.BlockSpec((EB0, DP), lambda i: (i, 0))
    blk = pl.BlockSpec((EB0, D), lambda i: (i, 0))
    return pl.pallas_call(
        kern,
        grid=(EPAD // EB0,),
        in_specs=[blkp, blkp, _full((1, GD)), _full((DP, D)), _full((GD, D)),
                  _full((1, D)), _full((D, D)), _full((1, D)), _full((D, DP)),
                  _full((1, DP)), _full((1, D)), _full((1, D))],
        out_specs=[blkp, blk],
        out_shape=[jax.ShapeDtypeStruct((EPAD, DP), f32),
                   jax.ShapeDtypeStruct((EPAD, D), f32)],
    )(sent, recv, g, wd, wg, b0.reshape(1, -1), w1, b1.reshape(1, -1),
      w2p, b2p, scale.reshape(1, -1), bias.reshape(1, -1))


def _tc_edge1(eprev, recv, nodes, g, mlp):
    (w0, b0), (w1, b1), (w2, b2) = mlp
    we, wd, wg = w0[:D], _rpad(w0[D:2 * D]), w0[2 * D:]
    w2p, b2p = _cpad(w2), _cpad(b2.reshape(1, -1))
    nb = EB1 // K

    def kern(e_r, r_r, n_r, g_r, we_r, wd_r, wg_r, b0_r, w1_r, b1_r,
             w2_r, b2_r, raw_o):
        sent = jnp.broadcast_to(n_r[...][:, None, :], (nb, K, DP)).reshape(EB1, DP)
        dif = sent - r_r[...]
        gv = jnp.dot(g_r[...], wg_r[...], preferred_element_type=f32) + b0_r[...]
        h = (jnp.dot(e_r[...], we_r[...], preferred_element_type=f32)
             + jnp.dot(dif, wd_r[...], preferred_element_type=f32) + gv)
        h = jnp.maximum(h, 0.0)
        h = jnp.maximum(jnp.dot(h, w1_r[...], preferred_element_type=f32) + b1_r[...], 0.0)
        raw_o[...] = jnp.dot(h, w2_r[...], preferred_element_type=f32) + b2_r[...]

    blkp = pl.BlockSpec((EB1, DP), lambda i: (i, 0))
    blk = pl.BlockSpec((EB1, D), lambda i: (i, 0))
    return pl.pallas_call(
        kern,
        grid=(E // EB1,),
        in_specs=[blk, blkp, pl.BlockSpec((nb, DP), lambda i: (i, 0)),
                  _full((1, GD)), _full((D, D)), _full((DP, D)), _full((GD, D)),
                  _full((1, D)), _full((D, D)), _full((1, D)), _full((D, DP)),
                  _full((1, DP))],
        out_specs=blkp,
        out_shape=jax.ShapeDtypeStruct((EPAD, DP), f32),
    )(eprev, recv, nodes, g, we, wd, wg, b0.reshape(1, -1), w1,
      b1.reshape(1, -1), w2p, b2p)


def _node_mlp_body(nodes, recvd, g_r, wn_r, wr_r, wg_r, b0_r, w1_r, b1_r,
                   w2_r, b2_r, sc_r, bi_r):
    gv = jnp.dot(g_r[...], wg_r[...], preferred_element_type=f32) + b0_r[...]
    h = (jnp.dot(nodes, wn_r[...], preferred_element_type=f32)
         + jnp.dot(recvd, wr_r[...], preferred_element_type=f32) + gv)
    h = jnp.maximum(h, 0.0)
    h = jnp.maximum(jnp.dot(h, w1_r[...], preferred_element_type=f32) + b1_r[...], 0.0)
    new = jnp.dot(h, w2_r[...], preferred_element_type=f32) + b2_r[...]
    x = nodes[:, :D] + new
    mu = jnp.mean(x, axis=-1, keepdims=True)
    var = jnp.mean((x - mu) ** 2, axis=-1, keepdims=True)
    return (x - mu) * lax.rsqrt(var + 1e-6) * sc_r[...] + bi_r[...]


def _tc_node(nodes, p0, p1, g, mlp, ln):
    (w0, b0), (w1, b1), (w2, b2) = mlp
    wn, wr, wg = _rpad(w0[:D]), _rpad(w0[D:2 * D]), w0[2 * D:]
    scale, bias = ln

    def kern(n_r, p0_r, p1_r, g_r, wn_r, wr_r, wg_r, b0_r, w1_r, b1_r,
             w2_r, b2_r, sc_r, bi_r, o_r):
        recvd = p0_r[...] + p1_r[...]
        y = _node_mlp_body(n_r[...], recvd, g_r, wn_r, wr_r, wg_r, b0_r,
                           w1_r, b1_r, w2_r, b2_r, sc_r, bi_r)
        o_r[...] = jnp.concatenate([y, jnp.zeros_like(y)], axis=-1)

    return pl.pallas_call(
        kern, out_shape=jax.ShapeDtypeStruct((N, DP), f32),
    )(nodes, p0, p1, g, wn, wr, wg, b0.reshape(1, -1), w1, b1.reshape(1, -1),
      w2, b2.reshape(1, -1), scale.reshape(1, -1), bias.reshape(1, -1))


def _tc_node_decode(nodes, p0, p1, g, mlp, ln, dec):
    (w0, b0), (w1, b1), (w2, b2) = mlp
    wn, wr, wg = _rpad(w0[:D]), _rpad(w0[D:2 * D]), w0[2 * D:]
    scale, bias = ln
    (dw0, db0), (dw1, db1), (dw2, db2) = dec

    def kern(n_r, p0_r, p1_r, g_r, wn_r, wr_r, wg_r, b0_r, w1_r, b1_r,
             w2_r, b2_r, sc_r, bi_r, dw0_r, db0_r, dw1_r, db1_r, dw2_r,
             db2_r, o_r):
        recvd = p0_r[...] + p1_r[...]
        x = _node_mlp_body(n_r[...], recvd, g_r, wn_r, wr_r, wg_r, b0_r,
                           w1_r, b1_r, w2_r, b2_r, sc_r, bi_r)
        h = jnp.maximum(jnp.dot(x, dw0_r[...], preferred_element_type=f32) + db0_r[...], 0.0)
        h = jnp.maximum(jnp.dot(h, dw1_r[...], preferred_element_type=f32) + db1_r[...], 0.0)
        o_r[...] = jnp.dot(h, dw2_r[...], preferred_element_type=f32) + db2_r[...]

    return pl.pallas_call(
        kern, out_shape=jax.ShapeDtypeStruct((N, 3), f32),
    )(nodes, p0, p1, g, wn, wr, wg, b0.reshape(1, -1), w1, b1.reshape(1, -1),
      w2, b2.reshape(1, -1), scale.reshape(1, -1), bias.reshape(1, -1),
      dw0, db0.reshape(1, -1), dw1, db1.reshape(1, -1), dw2, db2.reshape(1, -1))


NCHUNK = CPAD // CH
TOPT = 6


def _tc_knn(xpad):
    def kern(xa_r, xb_r, o_r):
        xa = xa_r[...]
        xb = xb_r[...]
        sqc = jnp.sum(xa * xa, axis=-1, keepdims=True)
        sqr = lax.dot_general(jnp.ones((1, DP), f32), xb * xb,
                              (((1,), (1,)), ((), ())),
                              preferred_element_type=f32)
        m = lax.dot_general(xa, xb, (((1,), (1,)), ((), ())),
                            preferred_element_type=f32)
        ri = lax.broadcasted_iota(i32, (CPAD, RB), 0)
        d2 = jnp.where(ri < N, sqc + sqr - 2.0 * m, BIGF)

        d3 = d2.reshape(NCHUNK, CH, RB)
        li = lax.broadcasted_iota(i32, (NCHUNK, CH, RB), 1)
        gb = lax.broadcasted_iota(i32, (NCHUNK, RB), 0) * CH
        ys, gs = [], []
        for _ in range(TOPT):
            mn = jnp.min(d3, axis=1)
            am = jnp.min(jnp.where(d3 <= mn[:, None, :], li, BIGI), axis=1)
            ys.append(mn)
            gs.append(gb + am)
            d3 = jnp.where(li == am[:, None, :], BIGF, d3)
        y = jnp.concatenate(ys, axis=0)
        gi = jnp.concatenate(gs, axis=0)

        last = 0
        for t in range(K):
            mn = jnp.min(y, axis=0, keepdims=True)
            ix = jnp.min(jnp.where(y <= mn, gi, BIGI), axis=0, keepdims=True)
            o_r[pl.ds(t, 1), :] = ix
            sel = gi == ix
            y = jnp.where(sel, BIGF, y)
            gi = jnp.where(sel, BIGI, gi)
        exhausted = jnp.any(gi[NCHUNK * (TOPT - 1):, :] == BIGI)

        @pl.when(exhausted)
        def _():
            dd = d2
            for t in range(K):
                mn = jnp.min(dd, axis=0, keepdims=True)
                ix = jnp.min(jnp.where(dd <= mn, ri, BIGI), axis=0,
                             keepdims=True)
                o_r[pl.ds(t, 1), :] = ix
                dd = jnp.where(ri == ix, BIGF, dd)

    return pl.pallas_call(
        kern,
        grid=(CPAD // RB,),
        in_specs=[_full((CPAD, DP)), pl.BlockSpec((RB, DP), lambda i: (i, 0))],
        out_specs=pl.BlockSpec((K, RB), lambda i: (0, i)),
        out_shape=jax.ShapeDtypeStruct((K, CPAD), i32),
    )(xpad, xpad)


def _pad_idx(idx, fill):
    idx = jnp.concatenate(
        [idx.astype(i32), jnp.full((EPAD - idx.shape[0],), fill, i32)])
    return idx.reshape(NW, NCH, CH)


def kernel(nodes, globals_, senders, receivers, params):
    g = globals_.reshape(1, GD)

    x = _tc_embed(nodes, params["embed_node"])

    sidx = _pad_idx(senders, 0)
    ridx = _pad_idx(receivers, 0)
    scat_idx0 = _pad_idx(receivers, N)
    sent = _sc_gather(x, sidx)
    recv = _sc_gather(x, ridx)
    e_raw, e_ln = _tc_edge0(sent, recv, g, params["edge_0"], params["ln_0"])
    parts = _sc_scatter_add(e_raw, scat_idx0)
    x = _tc_node(x, parts[0, :N], parts[1, :N], g, params["node_0"],
                 params["ln_0"])

    xpad = jnp.concatenate([x, jnp.zeros((CPAD - N, DP), f32)])
    knn = _tc_knn(xpad)[:, :N].T

    recv_flat = knn.reshape(-1)
    ridx1 = _pad_idx(recv_flat, 0)
    scat_idx1 = _pad_idx(recv_flat, N)
    recv1 = _sc_gather(x, ridx1)
    e1_raw = _tc_edge1(e_ln, recv1, x, g, params["edge_1"])
    parts1 = _sc_scatter_add(e1_raw, scat_idx1)
    return _tc_node_decode(x, parts1[0, :N], parts1[1, :N], g,
                           params["node_1"], params["ln_1"],
                           params["decode_node"])
